# baseline (speedup 1.0000x reference)
import functools

import jax
import jax.numpy as jnp
from jax import lax
from jax.experimental import pallas as pl
from jax.experimental.pallas import tpu as pltpu
from jax.experimental.pallas import tpu_sc as plsc

B = 16384
D = 64
NC = 2
NS = 16
NW = NC * NS
BPW = B // NW
L = 16

_mesh = plsc.VectorSubcoreMesh(core_axis_name="c", subcore_axis_name="s")


@functools.partial(
    pl.kernel,
    mesh=_mesh,
    out_type=jax.ShapeDtypeStruct((B, D), jnp.float32),
    compiler_params=pltpu.CompilerParams(use_tc_tiling_on_sc=False),
    scratch_types=[
        pltpu.VMEM((BPW,), jnp.int32),
        pltpu.VMEM((BPW, D), jnp.float32),
        pltpu.VMEM((BPW, D), jnp.float32),
        pltpu.SemaphoreType.DMA,
    ],
)
def _escorer(head_hbm, idx_hbm, table_hbm, out_hbm, idx_v, rows_v, head_v, sem):
    wid = lax.axis_index("s") * NC + lax.axis_index("c")
    base = wid * BPW
    pltpu.sync_copy(idx_hbm.at[pl.ds(base, BPW)], idx_v)
    gather = pltpu.async_copy(table_hbm.at[idx_v], rows_v, sem)
    pltpu.sync_copy(head_hbm.at[pl.ds(base, BPW), :], head_v)
    gather.wait()

    def body(r, carry):
        for j in range(D // L):
            sl = pl.ds(j * L, L)
            rows_v[r, sl] = rows_v[r, sl] + head_v[r, sl]
        return carry

    lax.fori_loop(0, BPW, body, 0, unroll=4)
    pltpu.sync_copy(rows_v, out_hbm.at[pl.ds(base, BPW), :])


def kernel(head_embed, rel_ids, embed_table):
    return _escorer(head_embed, rel_ids, embed_table)
